# baseline (speedup 1.0000x reference)
import functools
import math

import jax
import jax.numpy as jnp
from jax import lax
from jax.experimental import pallas as pl
from jax.experimental.pallas import tpu as pltpu
from jax.experimental.pallas import tpu_sc as plsc

B, S, H, D = 4, 2048, 16, 128
K = 32
T = 8192
BS = B * S
NIDX = BS * K

NC, NS = 2, 16
NW = NC * NS
NCH = 2
NIDX_C = NIDX // NCH
IDX_PER_W = NIDX_C // NW
CHUNK = 256
NPAIR = IDX_PER_W // (2 * CHUNK)
GPC = CHUNK // 128
IDX_ROWS_PER_W = IDX_PER_W // 128


def _sc_gather_body(ci, table_hbm, idx_hbm, out_hbm, idx_v, rows_a, rows_b,
                    gsem, wsem_a, wsem_b):
    wid = lax.axis_index("s") * NC + lax.axis_index("c")
    idx_base = ci * (NIDX_C // 128) + wid * IDX_ROWS_PER_W
    pltpu.sync_copy(idx_hbm.at[pl.ds(idx_base, IDX_ROWS_PER_W)], idx_v)
    out_base = wid * IDX_PER_W

    def fill(c, rows_v):
        cps = []
        for j in range(GPC):
            cps.append(
                pltpu.async_copy(
                    table_hbm.at[idx_v.at[c * GPC + j]],
                    rows_v.at[pl.ds(j * 128, 128)],
                    gsem,
                )
            )
        for cp in cps:
            cp.wait()

    def drain_write(rows_v, wsem):
        pltpu.make_async_copy(out_hbm.at[pl.ds(0, CHUNK)], rows_v, wsem).wait()

    def pair_body(t, carry):
        c0 = 2 * t

        @pl.when(t > 0)
        def _():
            drain_write(rows_a, wsem_a)

        fill(c0, rows_a)
        pltpu.async_copy(rows_a, out_hbm.at[pl.ds(out_base + c0 * CHUNK,
                                                  CHUNK)], wsem_a)

        @pl.when(t > 0)
        def _():
            drain_write(rows_b, wsem_b)

        fill(c0 + 1, rows_b)
        pltpu.async_copy(rows_b, out_hbm.at[pl.ds(out_base + (c0 + 1) * CHUNK,
                                                  CHUNK)], wsem_b)
        return carry

    lax.fori_loop(0, NPAIR, pair_body, 0)
    drain_write(rows_a, wsem_a)
    drain_write(rows_b, wsem_b)


@functools.lru_cache(maxsize=None)
def _sc_gather(ci):
    return functools.partial(
        pl.kernel,
        out_type=jax.ShapeDtypeStruct((NIDX_C, D // 2), jnp.int32),
        mesh=plsc.VectorSubcoreMesh(core_axis_name="c", subcore_axis_name="s"),
        scratch_types=[
            pltpu.VMEM((IDX_ROWS_PER_W, 128), jnp.int32),
            pltpu.VMEM((CHUNK, D // 2), jnp.int32),
            pltpu.VMEM((CHUNK, D // 2), jnp.int32),
            pltpu.SemaphoreType.DMA,
            pltpu.SemaphoreType.DMA,
            pltpu.SemaphoreType.DMA,
        ],
        compiler_params=pltpu.CompilerParams(use_tc_tiling_on_sc=False),
    )(functools.partial(_sc_gather_body, ci))


SUB = 8
POS_PER_STEP = 128
NSUB = POS_PER_STEP // SUB
PC = BS // NCH
GRID = PC // POS_PER_STEP
QR = POS_PER_STEP * H
KR = POS_PER_STEP * K
SCALE = 1.0 / math.sqrt(float(D))


def _attn_body(q_ref, kv_ref, o_prev_ref, o_ref, kvn_ref):
    del o_prev_ref
    dj = lax.broadcasted_iota(jnp.int32, (D // 2, D), 0)
    dd = lax.broadcasted_iota(jnp.int32, (D // 2, D), 1)
    p_e = (dd == 2 * dj).astype(jnp.bfloat16)
    p_o = (dd == 2 * dj + 1).astype(jnp.bfloat16)
    for sl in range(KR // 256):
        xw = kv_ref[pl.ds(sl * 256, 256), :]
        kv_e = lax.bitcast_convert_type(xw << 16, jnp.float32)
        kv_o = lax.bitcast_convert_type(xw & jnp.int32(-65536), jnp.float32)
        kvn = (
            lax.dot_general(kv_e.astype(jnp.bfloat16), p_e,
                            (((1,), (0,)), ((), ())),
                            preferred_element_type=jnp.float32)
            + lax.dot_general(kv_o.astype(jnp.bfloat16), p_o,
                              (((1,), (0,)), ((), ())),
                              preferred_element_type=jnp.float32)
        ).astype(jnp.bfloat16)
        kvn_ref[pl.ds(sl * 256, 256), :] = kvn
    rg = lax.broadcasted_iota(jnp.int32, (SUB * H, SUB * K), 0) // H
    cg = lax.broadcasted_iota(jnp.int32, (SUB * H, SUB * K), 1) // K
    mask = rg == cg
    for sb in range(NSUB):
        qs = q_ref[pl.ds(sb * SUB * H, SUB * H), :]
        kvs = kvn_ref[pl.ds(sb * SUB * K, SUB * K), :]
        logits = lax.dot_general(
            qs, kvs, (((1,), (1,)), ((), ())),
            preferred_element_type=jnp.float32,
        ) * SCALE
        l = jnp.where(mask, logits, -1e30)
        m = jnp.max(l, axis=1, keepdims=True)
        e = jnp.exp(l - m)
        s = jnp.sum(e, axis=1, keepdims=True) + 1e-9
        acc = lax.dot_general(
            e.astype(jnp.bfloat16), kvs, (((1,), (0,)), ((), ())),
            preferred_element_type=jnp.float32,
        )
        out = acc * (1.0 / s)
        o_ref[pl.ds(sb * SUB * H, SUB * H), :] = out.astype(jnp.bfloat16)


@functools.lru_cache(maxsize=None)
def _attn(ci):
    body = (_attn_body if ci
            else (lambda q, kv, o, kvn: _attn_body(q, kv, None, o, kvn)))
    in_specs = [
        pl.BlockSpec((QR, D), lambda i: (i + ci * GRID, 0)),
        pl.BlockSpec((KR, D // 2), lambda i: (i, 0)),
    ]
    if ci:
        in_specs.append(pl.BlockSpec(memory_space=pl.ANY))
    return pl.pallas_call(
        body,
        grid=(GRID,),
        in_specs=in_specs,
        out_specs=pl.BlockSpec((QR, D), lambda i: (i + ci * GRID, 0)),
        out_shape=jax.ShapeDtypeStruct((BS * H, D), jnp.bfloat16),
        input_output_aliases={2: 0} if ci else {},
        scratch_shapes=[pltpu.VMEM((KR, D), jnp.bfloat16)],
    )


def kernel(q, kv_flat, indices):
    idx = jnp.clip(indices, 0, T - 1).reshape(NIDX // 128, 128)
    table_f = lax.bitcast_convert_type(
        kv_flat.reshape(T, D // 2, 2), jnp.int32)
    q2 = q.reshape(BS * H, D)
    kv_gs = [_sc_gather(ci)(table_f, idx) for ci in range(NCH)]
    out = _attn(0)(q2, kv_gs[0])
    for ci in range(1, NCH):
        out = _attn(ci)(q2, kv_gs[ci], out)
    return out.reshape(B, S, H, D)


# speedup vs baseline: 1.5690x; 1.5690x over previous
import functools
import math

import jax
import jax.numpy as jnp
from jax import lax
from jax.experimental import pallas as pl
from jax.experimental.pallas import tpu as pltpu
from jax.experimental.pallas import tpu_sc as plsc

B, S, H, D = 4, 2048, 16, 128
K = 32
T = 8192
BS = B * S
NIDX = BS * K

NC, NS = 2, 16
NW = NC * NS
NCH = 2
NIDX_C = NIDX // NCH
IDX_PER_W = NIDX_C // NW
CHUNK = 256
NPAIR = IDX_PER_W // (2 * CHUNK)
GPC = CHUNK // 128
IDX_ROWS_PER_W = IDX_PER_W // 128


def _sc_gather_body(ci, table_hbm, idx_hbm, out_hbm, idx_v, rows_a, rows_b,
                    gsem, wsem_a, wsem_b):
    wid = lax.axis_index("s") * NC + lax.axis_index("c")
    idx_base = ci * (NIDX_C // 128) + wid * IDX_ROWS_PER_W
    pltpu.sync_copy(idx_hbm.at[pl.ds(idx_base, IDX_ROWS_PER_W)], idx_v)
    out_base = wid * IDX_PER_W

    def fill(c, rows_v):
        cps = []
        for j in range(GPC):
            cps.append(
                pltpu.async_copy(
                    table_hbm.at[idx_v.at[c * GPC + j]],
                    rows_v.at[pl.ds(j * 128, 128)],
                    gsem,
                )
            )
        for cp in cps:
            cp.wait()

    def drain_write(rows_v, wsem):
        pltpu.make_async_copy(out_hbm.at[pl.ds(0, CHUNK)], rows_v, wsem).wait()

    def pair_body(t, carry):
        c0 = 2 * t

        @pl.when(t > 0)
        def _():
            drain_write(rows_a, wsem_a)

        fill(c0, rows_a)
        pltpu.async_copy(rows_a, out_hbm.at[pl.ds(out_base + c0 * CHUNK,
                                                  CHUNK)], wsem_a)

        @pl.when(t > 0)
        def _():
            drain_write(rows_b, wsem_b)

        fill(c0 + 1, rows_b)
        pltpu.async_copy(rows_b, out_hbm.at[pl.ds(out_base + (c0 + 1) * CHUNK,
                                                  CHUNK)], wsem_b)
        return carry

    lax.fori_loop(0, NPAIR, pair_body, 0)
    drain_write(rows_a, wsem_a)
    drain_write(rows_b, wsem_b)


@functools.lru_cache(maxsize=None)
def _sc_gather(ci):
    return functools.partial(
        pl.kernel,
        out_type=jax.ShapeDtypeStruct((NIDX_C, D), jnp.float32),
        mesh=plsc.VectorSubcoreMesh(core_axis_name="c", subcore_axis_name="s"),
        scratch_types=[
            pltpu.VMEM((IDX_ROWS_PER_W, 128), jnp.int32),
            pltpu.VMEM((CHUNK, D), jnp.float32),
            pltpu.VMEM((CHUNK, D), jnp.float32),
            pltpu.SemaphoreType.DMA,
            pltpu.SemaphoreType.DMA,
            pltpu.SemaphoreType.DMA,
        ],
        compiler_params=pltpu.CompilerParams(use_tc_tiling_on_sc=False),
    )(functools.partial(_sc_gather_body, ci))


SUB = 8
POS_PER_STEP = 256
NSUB = POS_PER_STEP // SUB
PC = BS // NCH
GRID = PC // POS_PER_STEP
QR = POS_PER_STEP * H
KR = POS_PER_STEP * K
SCALE = 1.0 / math.sqrt(float(D))


def _attn_body(q_ref, kv_ref, o_prev_ref, o_ref):
    del o_prev_ref
    rg = lax.broadcasted_iota(jnp.int32, (SUB * H, SUB * K), 0) // H
    cg = lax.broadcasted_iota(jnp.int32, (SUB * H, SUB * K), 1) // K
    mask = rg == cg
    for sb in range(NSUB):
        qs = q_ref[pl.ds(sb * SUB * H, SUB * H), :]
        kvs = kv_ref[pl.ds(sb * SUB * K, SUB * K), :].astype(jnp.bfloat16)
        logits = lax.dot_general(
            qs, kvs, (((1,), (1,)), ((), ())),
            preferred_element_type=jnp.float32,
        ) * SCALE
        l = jnp.where(mask, logits, -1e30)
        m = jnp.max(l, axis=1, keepdims=True)
        e = jnp.exp(l - m)
        s = jnp.sum(e, axis=1, keepdims=True) + 1e-9
        acc = lax.dot_general(
            e.astype(jnp.bfloat16), kvs, (((1,), (0,)), ((), ())),
            preferred_element_type=jnp.float32,
        )
        out = acc * (1.0 / s)
        o_ref[pl.ds(sb * SUB * H, SUB * H), :] = out.astype(jnp.bfloat16)


@functools.lru_cache(maxsize=None)
def _attn(ci):
    body = _attn_body if ci else (lambda q, kv, o: _attn_body(q, kv, None, o))
    in_specs = [
        pl.BlockSpec((QR, D), lambda i: (i + ci * GRID, 0)),
        pl.BlockSpec((KR, D), lambda i: (i, 0)),
    ]
    if ci:
        in_specs.append(pl.BlockSpec(memory_space=pl.ANY))
    return pl.pallas_call(
        body,
        grid=(GRID,),
        in_specs=in_specs,
        out_specs=pl.BlockSpec((QR, D), lambda i: (i + ci * GRID, 0)),
        out_shape=jax.ShapeDtypeStruct((BS * H, D), jnp.bfloat16),
        input_output_aliases={2: 0} if ci else {},
    )


def kernel(q, kv_flat, indices):
    idx = jnp.clip(indices, 0, T - 1).reshape(NIDX // 128, 128)
    table_f = kv_flat.astype(jnp.float32)
    q2 = q.reshape(BS * H, D)
    kv_gs = [_sc_gather(ci)(table_f, idx) for ci in range(NCH)]
    out = _attn(0)(q2, kv_gs[0])
    for ci in range(1, NCH):
        out = _attn(ci)(q2, kv_gs[ci], out)
    return out.reshape(B, S, H, D)


# speedup vs baseline: 1.6174x; 1.0309x over previous
import functools
import math

import jax
import jax.numpy as jnp
from jax import lax
from jax.experimental import pallas as pl
from jax.experimental.pallas import tpu as pltpu
from jax.experimental.pallas import tpu_sc as plsc

B, S, H, D = 4, 2048, 16, 128
K = 32
T = 8192
BS = B * S
NIDX = BS * K

NC, NS = 2, 16
NW = NC * NS
NCH = 2
NIDX_C = NIDX // NCH
IDX_PER_W = NIDX_C // NW
CHUNK = 256
NPAIR = IDX_PER_W // (2 * CHUNK)
GPC = CHUNK // 128
IDX_ROWS_PER_W = IDX_PER_W // 128


def _sc_gather_body(ci, table_hbm, idx_hbm, out_hbm, idx_v, rows_a, rows_b,
                    gsem, wsem_a, wsem_b):
    wid = lax.axis_index("s") * NC + lax.axis_index("c")
    idx_base = ci * (NIDX_C // 128) + wid * IDX_ROWS_PER_W
    pltpu.sync_copy(idx_hbm.at[pl.ds(idx_base, IDX_ROWS_PER_W)], idx_v)
    out_base = wid * IDX_PER_W

    def fill(c, rows_v):
        cps = []
        for j in range(GPC):
            cps.append(
                pltpu.async_copy(
                    table_hbm.at[idx_v.at[c * GPC + j]],
                    rows_v.at[pl.ds(j * 128, 128)],
                    gsem,
                )
            )
        for cp in cps:
            cp.wait()

    def drain_write(rows_v, wsem):
        pltpu.make_async_copy(out_hbm.at[pl.ds(0, CHUNK)], rows_v, wsem).wait()

    def pair_body(t, carry):
        c0 = 2 * t

        @pl.when(t > 0)
        def _():
            drain_write(rows_a, wsem_a)

        fill(c0, rows_a)
        pltpu.async_copy(rows_a, out_hbm.at[pl.ds(out_base + c0 * CHUNK,
                                                  CHUNK)], wsem_a)

        @pl.when(t > 0)
        def _():
            drain_write(rows_b, wsem_b)

        fill(c0 + 1, rows_b)
        pltpu.async_copy(rows_b, out_hbm.at[pl.ds(out_base + (c0 + 1) * CHUNK,
                                                  CHUNK)], wsem_b)
        return carry

    lax.fori_loop(0, NPAIR, pair_body, 0)
    drain_write(rows_a, wsem_a)
    drain_write(rows_b, wsem_b)


@functools.lru_cache(maxsize=None)
def _sc_gather(ci):
    return functools.partial(
        pl.kernel,
        out_type=jax.ShapeDtypeStruct((NIDX_C, D), jnp.float32),
        mesh=plsc.VectorSubcoreMesh(core_axis_name="c", subcore_axis_name="s"),
        scratch_types=[
            pltpu.VMEM((IDX_ROWS_PER_W, 128), jnp.int32),
            pltpu.VMEM((CHUNK, D), jnp.float32),
            pltpu.VMEM((CHUNK, D), jnp.float32),
            pltpu.SemaphoreType.DMA,
            pltpu.SemaphoreType.DMA,
            pltpu.SemaphoreType.DMA,
        ],
        compiler_params=pltpu.CompilerParams(use_tc_tiling_on_sc=False),
    )(functools.partial(_sc_gather_body, ci))


SUB = 8
POS_PER_STEP = 512
NSUB = POS_PER_STEP // SUB
PC = BS // NCH
GRID = PC // POS_PER_STEP
QR = POS_PER_STEP * H
KR = POS_PER_STEP * K
SCALE = 1.0 / math.sqrt(float(D))


def _attn_body(q_ref, kv_ref, o_prev_ref, o_ref):
    del o_prev_ref
    rg = lax.broadcasted_iota(jnp.int32, (SUB * H, SUB * K), 0) // H
    cg = lax.broadcasted_iota(jnp.int32, (SUB * H, SUB * K), 1) // K
    mask = rg == cg
    for sb in range(NSUB):
        qs = q_ref[pl.ds(sb * SUB * H, SUB * H), :]
        kvs = kv_ref[pl.ds(sb * SUB * K, SUB * K), :].astype(jnp.bfloat16)
        logits = lax.dot_general(
            qs, kvs, (((1,), (1,)), ((), ())),
            preferred_element_type=jnp.float32,
        ) * SCALE
        l = jnp.where(mask, logits, -1e30)
        m = jnp.max(l, axis=1, keepdims=True)
        e = jnp.exp(l - m)
        s = jnp.sum(e, axis=1, keepdims=True) + 1e-9
        acc = lax.dot_general(
            e.astype(jnp.bfloat16), kvs, (((1,), (0,)), ((), ())),
            preferred_element_type=jnp.float32,
        )
        out = acc * (1.0 / s)
        o_ref[pl.ds(sb * SUB * H, SUB * H), :] = out.astype(jnp.bfloat16)


@functools.lru_cache(maxsize=None)
def _attn(ci):
    body = _attn_body if ci else (lambda q, kv, o: _attn_body(q, kv, None, o))
    in_specs = [
        pl.BlockSpec((QR, D), lambda i: (i + ci * GRID, 0)),
        pl.BlockSpec((KR, D), lambda i: (i, 0)),
    ]
    if ci:
        in_specs.append(pl.BlockSpec(memory_space=pl.ANY))
    return pl.pallas_call(
        body,
        grid=(GRID,),
        in_specs=in_specs,
        out_specs=pl.BlockSpec((QR, D), lambda i: (i + ci * GRID, 0)),
        out_shape=jax.ShapeDtypeStruct((BS * H, D), jnp.bfloat16),
        input_output_aliases={2: 0} if ci else {},
    )


def kernel(q, kv_flat, indices):
    idx = jnp.clip(indices, 0, T - 1).reshape(NIDX // 128, 128)
    table_f = kv_flat.astype(jnp.float32)
    q2 = q.reshape(BS * H, D)
    kv_gs = [_sc_gather(ci)(table_f, idx) for ci in range(NCH)]
    out = _attn(0)(q2, kv_gs[0])
    for ci in range(1, NCH):
        out = _attn(ci)(q2, kv_gs[ci], out)
    return out.reshape(B, S, H, D)
